# baseline (speedup 1.0000x reference)
import jax
import jax.numpy as jnp
from jax.experimental import pallas as pl

_BLOCK_ROWS = 4000


def _normalize_block(x_ref, o_ref):
    x = x_ref[...]
    w = x.shape[-1]
    h = w // 2
    x2 = (x * x).astype(jnp.bfloat16)
    row = jax.lax.broadcasted_iota(jnp.int32, (w, w), 0) // h
    col = jax.lax.broadcasted_iota(jnp.int32, (w, w), 1) // h
    blockdiag = (row == col).astype(jnp.bfloat16)
    s = jax.lax.dot_general(
        x2, blockdiag,
        dimension_numbers=(((1,), (0,)), ((), ())),
        preferred_element_type=jnp.float32,
    )
    o_ref[...] = x * jax.lax.rsqrt(jnp.maximum(s, 1e-24))


def kernel(prototypes):
    n, d = prototypes.shape
    xp = prototypes.reshape(n // 2, 2 * d)
    out = pl.pallas_call(
        _normalize_block,
        grid=(xp.shape[0] // _BLOCK_ROWS,),
        in_specs=[pl.BlockSpec((_BLOCK_ROWS, 2 * d), lambda i: (i, 0))],
        out_specs=pl.BlockSpec((_BLOCK_ROWS, 2 * d), lambda i: (i, 0)),
        out_shape=jax.ShapeDtypeStruct(xp.shape, xp.dtype),
    )(xp)
    return out.reshape(n, d)


# speedup vs baseline: 7.6744x; 7.6744x over previous
import jax
import jax.numpy as jnp
from jax.experimental import pallas as pl

_BLOCK_COLS = 8192


def _normalize_block(x_ref, o_ref):
    x = x_ref[...]
    s = jnp.sum(x * x, axis=0, keepdims=True)
    o_ref[...] = x * jax.lax.rsqrt(jnp.maximum(s, 1e-24))


def kernel(prototypes):
    n, d = prototypes.shape
    xt = prototypes.T
    out_t = pl.pallas_call(
        _normalize_block,
        grid=(pl.cdiv(n, _BLOCK_COLS),),
        in_specs=[pl.BlockSpec((d, _BLOCK_COLS), lambda i: (0, i))],
        out_specs=pl.BlockSpec((d, _BLOCK_COLS), lambda i: (0, i)),
        out_shape=jax.ShapeDtypeStruct((d, n), prototypes.dtype),
    )(xt)
    return out_t.T


# speedup vs baseline: 8.7972x; 1.1463x over previous
import jax
import jax.numpy as jnp
from jax.experimental import pallas as pl

_BLOCK_COLS = 32768


def _normalize_block(x_ref, o_ref):
    x = x_ref[...]
    s = jnp.sum(x * x, axis=0, keepdims=True)
    o_ref[...] = x * jax.lax.rsqrt(jnp.maximum(s, 1e-24))


def kernel(prototypes):
    n, d = prototypes.shape
    xt = prototypes.T
    out_t = pl.pallas_call(
        _normalize_block,
        grid=(pl.cdiv(n, _BLOCK_COLS),),
        in_specs=[pl.BlockSpec((d, _BLOCK_COLS), lambda i: (0, i))],
        out_specs=pl.BlockSpec((d, _BLOCK_COLS), lambda i: (0, i)),
        out_shape=jax.ShapeDtypeStruct((d, n), prototypes.dtype),
    )(xt)
    return out_t.T


# speedup vs baseline: 8.8305x; 1.0038x over previous
import jax
import jax.numpy as jnp
from jax.experimental import pallas as pl

_BLOCK_COLS = 49152


def _normalize_block(x_ref, o_ref):
    x = x_ref[...]
    s = jnp.sum(x * x, axis=0, keepdims=True)
    o_ref[...] = x * jax.lax.rsqrt(jnp.maximum(s, 1e-24))


def kernel(prototypes):
    n, d = prototypes.shape
    xt = prototypes.T
    out_t = pl.pallas_call(
        _normalize_block,
        grid=(pl.cdiv(n, _BLOCK_COLS),),
        in_specs=[pl.BlockSpec((d, _BLOCK_COLS), lambda i: (0, i))],
        out_specs=pl.BlockSpec((d, _BLOCK_COLS), lambda i: (0, i)),
        out_shape=jax.ShapeDtypeStruct((d, n), prototypes.dtype),
    )(xt)
    return out_t.T
